# Initial kernel scaffold; baseline (speedup 1.0000x reference)
import jax
import jax.numpy as jnp
from jax.experimental import pallas as pl


def kernel(x, ft_mask, W_ft, b_ft, Wl1, Wr1, a1, Wl2, Wr2, a2, Wl3, Wr3, a3, Wl4, Wr4, a4, W1, b1, W2, b2, Wg, bg, geod_v, geod_scale):
    raise NotImplementedError("write your pallas kernel here")


# speedup vs baseline: 5.2699x; 5.2699x over previous
import functools

import jax
import jax.numpy as jnp
from jax import lax
from jax.experimental import pallas as pl
from jax.experimental.pallas import tpu as pltpu

NV = 2048
NB = 2
KNN = 16
_PREC = lax.Precision.DEFAULT
_ROWS = 256


def _ft_body(x_ref, m_ref, w_ref, b_ref, o_ref):
    xm = x_ref[...] * m_ref[0]
    o_ref[0] = jnp.dot(xm, w_ref[0], precision=_PREC) + b_ref[0]


def _ft(x, ft_mask, W_ft, b_ft):
    return pl.pallas_call(
        _ft_body,
        grid=(NB,),
        in_specs=[
            pl.BlockSpec((NV, 16), lambda b: (0, 0)),
            pl.BlockSpec((1, 1, 16), lambda b: (b, 0, 0)),
            pl.BlockSpec((1, 16, 256), lambda b: (b, 0, 0)),
            pl.BlockSpec((1, 1, 256), lambda b: (b, 0, 0)),
        ],
        out_specs=pl.BlockSpec((1, NV, 256), lambda b: (b, 0, 0)),
        out_shape=jax.ShapeDtypeStruct((NB, NV, 256), jnp.float32),
    )(x, ft_mask.reshape(NB, 1, 16), W_ft, b_ft.reshape(NB, 1, 256))


def _pre_body(hfull_ref, hrow_ref, wl_ref, wr_ref, gl_ref, gr_ref, idx_ref):
    b = pl.program_id(0)
    rows = hrow_ref[0]
    hf = hfull_ref[0]
    gl_ref[0] = jnp.dot(rows, wl_ref[...], precision=_PREC)
    gr_ref[0] = jnp.dot(rows, wr_ref[...], precision=_PREC)
    sqf = jnp.sum(hf * hf, axis=-1)
    sqr = jnp.sum(rows * rows, axis=-1)
    mm = lax.dot_general(rows, hf, (((1,), (1,)), ((), ())), precision=_PREC)
    d = (sqr[:, None] + sqf[None, :]) - 2.0 * mm
    cols = lax.broadcasted_iota(jnp.int32, d.shape, 1)
    picks = []
    for _ in range(KNN):
        mn = jnp.min(d, axis=1, keepdims=True)
        ij = jnp.min(jnp.where(d == mn, cols, NV), axis=1, keepdims=True)
        picks.append(ij)
        d = jnp.where(cols == ij, jnp.float32(jnp.inf), d)
    idx_ref[0] = jnp.concatenate(picks, axis=1) + b * NV


def _pre(h, Wl, Wr):
    din, dout = Wl.shape
    nblk = NV // _ROWS
    return pl.pallas_call(
        _pre_body,
        grid=(NB, nblk),
        in_specs=[
            pl.BlockSpec((1, NV, din), lambda b, r: (b, 0, 0)),
            pl.BlockSpec((1, _ROWS, din), lambda b, r: (b, r, 0)),
            pl.BlockSpec((din, dout), lambda b, r: (0, 0)),
            pl.BlockSpec((din, dout), lambda b, r: (0, 0)),
        ],
        out_specs=[
            pl.BlockSpec((1, _ROWS, dout), lambda b, r: (b, r, 0)),
            pl.BlockSpec((1, _ROWS, dout), lambda b, r: (b, r, 0)),
            pl.BlockSpec((1, _ROWS, KNN), lambda b, r: (b, r, 0)),
        ],
        out_shape=[
            jax.ShapeDtypeStruct((NB, NV, dout), jnp.float32),
            jax.ShapeDtypeStruct((NB, NV, dout), jnp.float32),
            jax.ShapeDtypeStruct((NB, NV, KNN), jnp.int32),
        ],
    )(h, h, Wl, Wr)


def _att(gl, gr, idxg, a):
    nbr = gr[idxg]
    z = jax.nn.leaky_relu(gl[:, None, :] + nbr, negative_slope=0.2)
    e = jnp.einsum('nke,e->nk', z, a)
    alpha = jax.nn.softmax(e, axis=-1)
    return jnp.sum(alpha[..., None] * nbr, axis=1)


def _gat_layer(h, Wl, Wr, a):
    dout = Wl.shape[1]
    gl, gr, idx = _pre(h, Wl, Wr)
    o = _att(gl.reshape(NB * NV, dout), gr.reshape(NB * NV, dout),
             idx.reshape(NB * NV, KNN), a)
    return o.reshape(NB, NV, dout)


def _mlp_body(m_ref, w1_ref, b1_ref, w2_ref, b2_ref, wg_ref, bg_ref, gv_ref, o_ref):
    h = jnp.maximum(jnp.dot(m_ref[...], w1_ref[...], precision=_PREC) + b1_ref[...], 0.0)
    h = jnp.maximum(jnp.dot(h, w2_ref[...], precision=_PREC) + b2_ref[...], 0.0)
    y = jnp.tanh(jnp.dot(h, wg_ref[...], precision=_PREC) + bg_ref[...])
    o_ref[...] = y * gv_ref[...]


def _mlp(merged, W1, b1, W2, b2, Wg_s, bg_s, geod_v):
    return pl.pallas_call(
        _mlp_body,
        out_shape=jax.ShapeDtypeStruct((NV, 3), jnp.float32),
    )(merged, W1, b1.reshape(1, 256), W2, b2.reshape(1, 64),
      Wg_s, bg_s.reshape(1, 3), geod_v.reshape(NV, 1))


def kernel(x, ft_mask, W_ft, b_ft, Wl1, Wr1, a1, Wl2, Wr2, a2, Wl3, Wr3, a3,
           Wl4, Wr4, a4, W1, b1, W2, b2, Wg, bg, geod_v, geod_scale):
    h0 = _ft(x, ft_mask, W_ft, b_ft)
    o1 = _gat_layer(h0, Wl1, Wr1, a1)
    o2 = _gat_layer(jnp.concatenate([h0, o1], axis=-1), Wl2, Wr2, a2)
    o3 = _gat_layer(jnp.concatenate([o1, o2], axis=-1), Wl3, Wr3, a3)
    o4 = _gat_layer(jnp.concatenate([o2, o3], axis=-1), Wl4, Wr4, a4)
    merged = jnp.concatenate([o4[0], o4[1]], axis=-1)
    return _mlp(merged, W1, b1, W2, b2, Wg * geod_scale, bg * geod_scale, geod_v)
